# Initial kernel scaffold; baseline (speedup 1.0000x reference)
import jax
import jax.numpy as jnp
from jax.experimental import pallas as pl


def kernel(x, edge_index, edge_in, in_w, edge_out, out_w, W1, bias1, Wc, bc):
    raise NotImplementedError("write your pallas kernel here")


# speedup vs baseline: 3.8619x; 3.8619x over previous
import jax
import jax.numpy as jnp
from jax.experimental import pallas as pl

N = 100000
OUT = 32


def _gcn_prop(h, edge_index, edge_weight):
    row = edge_index[0]
    col = edge_index[1]
    e = row.shape[0]
    if edge_weight is None:
        edge_weight = jnp.ones((e,), dtype=h.dtype)
    w = jnp.where(row == col, jnp.zeros_like(edge_weight), edge_weight)
    deg = jax.ops.segment_sum(w, col, num_segments=N) + 1.0
    dis = deg ** -0.5
    g = dis[:, None] * h
    msg = w[:, None] * g[row]
    acc = jax.ops.segment_sum(msg, col, num_segments=N) + g
    return dis[:, None] * acc


def _final_body(cat_ref, wc_ref, bc_ref, o_ref):
    o_ref[...] = cat_ref[...] @ wc_ref[...] + bc_ref[...]


def kernel(x, edge_index, edge_in, in_w, edge_out, out_w, W1, bias1, Wc, bc):
    h = x @ W1
    x1 = _gcn_prop(h, edge_index, None) + bias1
    x2 = _gcn_prop(h, edge_in, in_w) + bias1
    x3 = _gcn_prop(h, edge_out, out_w) + bias1
    cat = jnp.concatenate([x1, x2, x3], axis=-1)
    npad = 100352
    catp = jnp.pad(cat, ((0, npad - N), (0, 0)))
    blk = 2048
    out = pl.pallas_call(
        _final_body,
        grid=(npad // blk,),
        in_specs=[
            pl.BlockSpec((blk, 3 * OUT), lambda i: (i, 0)),
            pl.BlockSpec((3 * OUT, OUT), lambda i: (0, 0)),
            pl.BlockSpec((1, OUT), lambda i: (0, 0)),
        ],
        out_specs=pl.BlockSpec((blk, OUT), lambda i: (i, 0)),
        out_shape=jax.ShapeDtypeStruct((npad, OUT), jnp.float32),
    )(catp, Wc.T, bc[None, :])
    return out[:N]


# speedup vs baseline: 18.2021x; 4.7133x over previous
import functools

import jax
import jax.numpy as jnp
from jax import lax
from jax.experimental import pallas as pl
from jax.experimental.pallas import tpu as pltpu
from jax.experimental.pallas import tpu_sc as plsc

N = 100000
E = 1600000
IN_DIM = 128
OUT = 32
H = 16
NS = 16
NPAD = 100352
RPT = NPAD // NS
RCH = 896
ECH = 1024
NCH = 98
EPAD = NS * ECH * NCH
SUB = 128


def _rsqrt16(d):
    s = jnp.float32(0.25) * d + jnp.float32(1.0)
    for _ in range(8):
        s = jnp.float32(0.5) * (s + d / s)
    return jnp.float32(1.0) / s


def _sc_body(h2, rall, call, wall,
             cat6, g,
             acc, deg,
             rbuf, cbuf, wbuf, gidx, sidx, wsub, grows, dense, dslice):
    c = lax.axis_index("c")
    s = lax.axis_index("s")
    row0 = s * RPT
    goff = c * NPAD

    def _stage_edges(st):
        pltpu.sync_copy(rall.at[pl.ds(st, ECH)], rbuf)
        pltpu.sync_copy(call.at[pl.ds(st, ECH)], cbuf)
        pltpu.sync_copy(wall.at[pl.ds(st, ECH)], wbuf)

    def _edge_group(off, need_gidx):
        for gq in range(SUB // 16):
            o = off + gq * 16
            rv = rbuf[pl.ds(o, 16)]
            cv = cbuf[pl.ds(o, 16)]
            wv = wbuf[pl.ds(o, 16)]
            w1 = jnp.where(rv == cv, jnp.float32(0.0), wv)
            wsub[pl.ds(gq * 16, 16)] = w1
            sidx[pl.ds(gq * 16, 16)] = cv
            if need_gidx:
                gidx[pl.ds(gq * 16, 16)] = rv + goff

    def _set(k, _):
        ebase = k * EPAD + s * (ECH * NCH)

        for j in range(RCH // 16):
            dslice[pl.ds(j * 16, 16)] = jnp.full((16,), 1.0, jnp.float32)

        def _p0(q, _):
            pltpu.sync_copy(dslice, deg.at[pl.ds(row0 + q * RCH, RCH)])
            return 0
        lax.fori_loop(0, RPT // RCH, _p0, 0)
        plsc.subcore_barrier()

        def _p1(i, _):
            _stage_edges(ebase + i * ECH)

            def _p1s(sub, _):
                _edge_group(sub * SUB, False)
                pltpu.sync_copy(wsub, deg.at[sidx], add=True)
                return 0
            lax.fori_loop(0, ECH // SUB, _p1s, 0)
            return 0
        lax.fori_loop(0, NCH, _p1, 0)
        plsc.subcore_barrier()

        def _p2(q, _):
            r0 = row0 + q * RCH
            pltpu.sync_copy(deg.at[pl.ds(r0, RCH)], dslice)
            for j in range(RCH // 16):
                d = dslice[pl.ds(j * 16, 16)]
                dslice[pl.ds(j * 16, 16)] = _rsqrt16(d)
            pltpu.sync_copy(dslice, deg.at[pl.ds(r0, RCH)])
            pltpu.sync_copy(h2.at[c, pl.ds(r0, RCH), :], dense)

            def _rs(m, _):
                dv = dslice[pl.ds(m * 16, 16)]
                for j in range(16):
                    n = m * 16 + j
                    dense[n, :] = dense[n, :] * dv[j]
                return 0
            lax.fori_loop(0, RCH // 16, _rs, 0)
            pltpu.sync_copy(dense, g.at[pl.ds(goff + r0, RCH), :])
            pltpu.sync_copy(dense, acc.at[pl.ds(r0, RCH), :])
            return 0
        lax.fori_loop(0, RPT // RCH, _p2, 0)
        plsc.subcore_barrier()

        def _p3(i, _):
            _stage_edges(ebase + i * ECH)

            def _p3s(sub, _):
                _edge_group(sub * SUB, True)
                pltpu.sync_copy(g.at[gidx], grows)

                def _sc(q, _):
                    wv = wsub[pl.ds(q * 16, 16)]
                    for u in range(16):
                        b = q * 16 + u
                        grows[b, :] = grows[b, :] * wv[u]
                    return 0
                lax.fori_loop(0, SUB // 16, _sc, 0)
                pltpu.sync_copy(grows, acc.at[sidx], add=True)
                return 0
            lax.fori_loop(0, ECH // SUB, _p3s, 0)
            return 0
        lax.fori_loop(0, NCH, _p3, 0)
        plsc.subcore_barrier()

        def _p4(q, _):
            r0 = row0 + q * RCH
            pltpu.sync_copy(acc.at[pl.ds(r0, RCH), :], dense)
            pltpu.sync_copy(deg.at[pl.ds(r0, RCH)], dslice)

            def _rs(m, _):
                dv = dslice[pl.ds(m * 16, 16)]
                for j in range(16):
                    n = m * 16 + j
                    dense[n, :] = dense[n, :] * dv[j]
                return 0
            lax.fori_loop(0, RCH // 16, _rs, 0)
            pltpu.sync_copy(dense, cat6.at[k, c, pl.ds(r0, RCH), :])
            return 0
        lax.fori_loop(0, RPT // RCH, _p4, 0)
        plsc.subcore_barrier()
        return 0

    lax.fori_loop(0, 3, _set, 0)


def _sc_call(h2, rall, call, wall):
    mesh = plsc.VectorSubcoreMesh(core_axis_name="c", subcore_axis_name="s")
    f = pl.kernel(
        _sc_body,
        out_type=(jax.ShapeDtypeStruct((3, 2, NPAD, H), jnp.float32),
                  jax.ShapeDtypeStruct((2 * NPAD, H), jnp.float32)),
        mesh=mesh,
        compiler_params=pltpu.CompilerParams(use_tc_tiling_on_sc=False),
        scratch_types=[
            pltpu.VMEM_SHARED((NPAD, H), jnp.float32),
            pltpu.VMEM_SHARED((NPAD,), jnp.float32),
            pltpu.VMEM((ECH,), jnp.int32),
            pltpu.VMEM((ECH,), jnp.int32),
            pltpu.VMEM((ECH,), jnp.float32),
            pltpu.VMEM((SUB,), jnp.int32),
            pltpu.VMEM((SUB,), jnp.int32),
            pltpu.VMEM((SUB,), jnp.float32),
            pltpu.VMEM((SUB, H), jnp.float32),
            pltpu.VMEM((RCH, H), jnp.float32),
            pltpu.VMEM((RCH,), jnp.float32),
        ],
    )
    return f(h2, rall, call, wall)


def _mm1_body(x_ref, w_ref, o_ref):
    o_ref[...] = jnp.dot(x_ref[...], w_ref[...],
                         preferred_element_type=jnp.float32)


def _mm1(xp, W1):
    blk = 2048
    return pl.pallas_call(
        _mm1_body,
        grid=(NPAD // blk,),
        in_specs=[pl.BlockSpec((blk, IN_DIM), lambda i: (i, 0)),
                  pl.BlockSpec((IN_DIM, OUT), lambda i: (0, 0))],
        out_specs=pl.BlockSpec((blk, OUT), lambda i: (i, 0)),
        out_shape=jax.ShapeDtypeStruct((NPAD, OUT), jnp.float32),
    )(xp, W1)


def _fin_body(cat_ref, w_ref, b_ref, o_ref):
    j = pl.program_id(1)

    @pl.when(j == 0)
    def _():
        o_ref[...] = jnp.broadcast_to(b_ref[...], o_ref.shape)

    o_ref[...] += jnp.dot(cat_ref[0, 0], w_ref[0, 0],
                          preferred_element_type=jnp.float32)


def _final(cat6, WcT6, bc2):
    blk = 2048
    return pl.pallas_call(
        _fin_body,
        grid=(NPAD // blk, 6),
        in_specs=[pl.BlockSpec((1, 1, blk, H), lambda i, j: (j // 2, j % 2, i, 0)),
                  pl.BlockSpec((1, 1, H, OUT), lambda i, j: (j // 2, j % 2, 0, 0)),
                  pl.BlockSpec((1, OUT), lambda i, j: (0, 0))],
        out_specs=pl.BlockSpec((blk, OUT), lambda i, j: (i, 0)),
        out_shape=jax.ShapeDtypeStruct((NPAD, OUT), jnp.float32),
    )(cat6, WcT6, bc2)


def kernel(x, edge_index, edge_in, in_w, edge_out, out_w, W1, bias1, Wc, bc):
    xp = jnp.pad(x, ((0, NPAD - N), (0, 0)))
    h = _mm1(xp, W1)
    h2 = jnp.stack([h[:, :H], h[:, H:]])

    ez = jnp.zeros((EPAD - E,), jnp.int32)
    wz = jnp.zeros((EPAD - E,), jnp.float32)
    rall = jnp.concatenate([edge_index[0], ez, edge_in[0], ez, edge_out[0], ez])
    call = jnp.concatenate([edge_index[1], ez, edge_in[1], ez, edge_out[1], ez])
    wall = jnp.concatenate([jnp.ones((E,), jnp.float32), wz, in_w, wz, out_w, wz])

    cat6, _ = _sc_call(h2, rall, call, wall)

    WcT6 = Wc.T.reshape(3, 2, H, OUT)
    bc2 = (bc + jnp.tile(bias1[0], 3) @ Wc.T)[None, :]
    out = _final(cat6, WcT6, bc2)
    return out[:N]


# speedup vs baseline: 29.8405x; 1.6394x over previous
import functools

import jax
import jax.numpy as jnp
from jax import lax
from jax.experimental import pallas as pl
from jax.experimental.pallas import tpu as pltpu
from jax.experimental.pallas import tpu_sc as plsc

N = 100000
E = 1600000
IN_DIM = 128
OUT = 32
H = 16
NS = 16
NPAD = 100352
RPT = NPAD // NS
RCH = 128
ECH = 256
NCH = 392
EPAD = NS * ECH * NCH
SUB = 128


def _rsqrt16(d):
    s = jnp.float32(0.25) * d + jnp.float32(1.0)
    for _ in range(8):
        s = jnp.float32(0.5) * (s + d / s)
    return jnp.float32(1.0) / s


def _sc_body(h2, rall, call, wall,
             cat6, g,
             acc, deg,
             ebuf_r, ebuf_c, ebuf_w,
             gidx2, sidx2, wsub2,
             grows2,
             dense, dslice,
             semstage, semg, sems, semd):
    c = lax.axis_index("c")
    s = lax.axis_index("s")
    row0 = s * RPT
    goff = c * NPAD
    NSUB = ECH // SUB

    def _issue_stage(st, b):
        pltpu.async_copy(rall.at[pl.ds(st, ECH)], ebuf_r.at[b], semstage)
        pltpu.async_copy(call.at[pl.ds(st, ECH)], ebuf_c.at[b], semstage)
        pltpu.async_copy(wall.at[pl.ds(st, ECH)], ebuf_w.at[b], semstage)

    def _drain_stage():
        pltpu.make_async_copy(rall.at[pl.ds(0, ECH)], ebuf_r.at[0], semstage).wait()
        pltpu.make_async_copy(call.at[pl.ds(0, ECH)], ebuf_c.at[0], semstage).wait()
        pltpu.make_async_copy(wall.at[pl.ds(0, ECH)], ebuf_w.at[0], semstage).wait()

    def _compute_idx(b, sub, need_gidx):
        for gq in range(SUB // 16):
            o = sub * SUB + gq * 16
            rv = ebuf_r[b, pl.ds(o, 16)]
            cv = ebuf_c[b, pl.ds(o, 16)]
            wv = ebuf_w[b, pl.ds(o, 16)]
            w1 = jnp.where(rv == cv, jnp.float32(0.0), wv)
            wsub2[b, sub, pl.ds(gq * 16, 16)] = w1
            sidx2[b, sub, pl.ds(gq * 16, 16)] = cv
            if need_gidx:
                gidx2[b, sub, pl.ds(gq * 16, 16)] = rv + goff

    def _drain_deg_scatters(p):
        for _ in range(NSUB):
            pltpu.make_async_copy(wsub2.at[0, 0], deg.at[sidx2.at[0, 0]],
                                  semd.at[p]).wait()

    def _drain_gathers(p):
        for _ in range(NSUB):
            pltpu.make_async_copy(g.at[gidx2.at[0, 0]], grows2.at[0, 0],
                                  semg.at[p]).wait()

    def _drain_acc_scatters(p):
        for _ in range(NSUB):
            pltpu.make_async_copy(grows2.at[0, 0], acc.at[sidx2.at[0, 0]],
                                  sems.at[p]).wait()

    def _set(k, _):
        ebase = k * EPAD + s * (ECH * NCH)

        for j in range(RCH // 16):
            dslice[pl.ds(j * 16, 16)] = jnp.full((16,), 1.0, jnp.float32)

        def _p0(q, _):
            pltpu.sync_copy(dslice, deg.at[pl.ds(row0 + q * RCH, RCH)])
            return 0
        lax.fori_loop(0, RPT // RCH, _p0, 0)
        plsc.subcore_barrier()

        _issue_stage(ebase, 0)

        def _p1(i, _):
            b = i % 2
            _drain_stage()

            @pl.when(i + 1 < NCH)
            def _():
                _issue_stage(ebase + (i + 1) * ECH, 1 - b)

            @pl.when(i >= 2)
            def _():
                _drain_deg_scatters(b)

            def _p1s(sub, _):
                _compute_idx(b, sub, False)
                pltpu.async_copy(wsub2.at[b, sub], deg.at[sidx2.at[b, sub]],
                                 semd.at[b], add=True)
                return 0
            lax.fori_loop(0, NSUB, _p1s, 0)
            return 0
        lax.fori_loop(0, NCH, _p1, 0)
        _drain_deg_scatters(0)
        _drain_deg_scatters(1)
        plsc.subcore_barrier()

        def _p2(q, _):
            r0 = row0 + q * RCH
            pltpu.sync_copy(deg.at[pl.ds(r0, RCH)], dslice)
            for j in range(RCH // 16):
                d = dslice[pl.ds(j * 16, 16)]
                dslice[pl.ds(j * 16, 16)] = _rsqrt16(d)
            pltpu.sync_copy(dslice, deg.at[pl.ds(r0, RCH)])
            pltpu.sync_copy(h2.at[c, pl.ds(r0, RCH), :], dense)

            def _rs(m, _):
                dv = dslice[pl.ds(m * 16, 16)]
                for j in range(16):
                    n = m * 16 + j
                    dense[n, :] = dense[n, :] * dv[j]
                return 0
            lax.fori_loop(0, RCH // 16, _rs, 0)
            pltpu.sync_copy(dense, g.at[pl.ds(goff + r0, RCH), :])
            pltpu.sync_copy(dense, acc.at[pl.ds(r0, RCH), :])
            return 0
        lax.fori_loop(0, RPT // RCH, _p2, 0)
        plsc.subcore_barrier()

        _issue_stage(ebase, 0)

        def _p3(i, _):
            b = i % 2

            @pl.when(i < NCH)
            def _():
                _drain_stage()

                @pl.when(i + 1 < NCH)
                def _():
                    _issue_stage(ebase + (i + 1) * ECH, 1 - b)

                @pl.when(i >= 2)
                def _():
                    _drain_acc_scatters(b)

                def _p3a(sub, _):
                    _compute_idx(b, sub, True)
                    pltpu.async_copy(g.at[gidx2.at[b, sub]],
                                     grows2.at[b, sub], semg.at[b])
                    return 0
                lax.fori_loop(0, NSUB, _p3a, 0)

            @pl.when(i > 0)
            def _():
                pb = 1 - b
                _drain_gathers(pb)

                def _p3b(sub, _):
                    def _sc(q, _):
                        wv = wsub2[pb, sub, pl.ds(q * 16, 16)]
                        for u in range(16):
                            brow = q * 16 + u
                            grows2[pb, sub, brow, :] = (
                                grows2[pb, sub, brow, :] * wv[u])
                        return 0
                    lax.fori_loop(0, SUB // 16, _sc, 0)
                    pltpu.async_copy(grows2.at[pb, sub],
                                     acc.at[sidx2.at[pb, sub]],
                                     sems.at[pb], add=True)
                    return 0
                lax.fori_loop(0, NSUB, _p3b, 0)
            return 0
        lax.fori_loop(0, NCH + 1, _p3, 0)
        _drain_acc_scatters(0)
        _drain_acc_scatters(1)
        plsc.subcore_barrier()

        def _p4(q, _):
            r0 = row0 + q * RCH
            pltpu.sync_copy(acc.at[pl.ds(r0, RCH), :], dense)
            pltpu.sync_copy(deg.at[pl.ds(r0, RCH)], dslice)

            def _rs(m, _):
                dv = dslice[pl.ds(m * 16, 16)]
                for j in range(16):
                    n = m * 16 + j
                    dense[n, :] = dense[n, :] * dv[j]
                return 0
            lax.fori_loop(0, RCH // 16, _rs, 0)
            pltpu.sync_copy(dense, cat6.at[k, c, pl.ds(r0, RCH), :])
            return 0
        lax.fori_loop(0, RPT // RCH, _p4, 0)
        plsc.subcore_barrier()
        return 0

    lax.fori_loop(0, 3, _set, 0)


def _sc_call(h2, rall, call, wall):
    mesh = plsc.VectorSubcoreMesh(core_axis_name="c", subcore_axis_name="s")
    f = pl.kernel(
        _sc_body,
        out_type=(jax.ShapeDtypeStruct((3, 2, NPAD, H), jnp.float32),
                  jax.ShapeDtypeStruct((2 * NPAD, H), jnp.float32)),
        mesh=mesh,
        compiler_params=pltpu.CompilerParams(use_tc_tiling_on_sc=False),
        scratch_types=[
            pltpu.VMEM_SHARED((NPAD, H), jnp.float32),
            pltpu.VMEM_SHARED((NPAD,), jnp.float32),
            pltpu.VMEM((2, ECH), jnp.int32),
            pltpu.VMEM((2, ECH), jnp.int32),
            pltpu.VMEM((2, ECH), jnp.float32),
            pltpu.VMEM((2, ECH // SUB, SUB), jnp.int32),
            pltpu.VMEM((2, ECH // SUB, SUB), jnp.int32),
            pltpu.VMEM((2, ECH // SUB, SUB), jnp.float32),
            pltpu.VMEM((2, ECH // SUB, SUB, H), jnp.float32),
            pltpu.VMEM((RCH, H), jnp.float32),
            pltpu.VMEM((RCH,), jnp.float32),
            pltpu.SemaphoreType.DMA,
            pltpu.SemaphoreType.DMA((2,)),
            pltpu.SemaphoreType.DMA((2,)),
            pltpu.SemaphoreType.DMA((2,)),
        ],
    )
    return f(h2, rall, call, wall)


def _mm1_body(x_ref, w_ref, o_ref):
    o_ref[...] = jnp.dot(x_ref[...], w_ref[...],
                         preferred_element_type=jnp.float32)


def _mm1(xp, W1):
    blk = 2048
    return pl.pallas_call(
        _mm1_body,
        grid=(NPAD // blk,),
        in_specs=[pl.BlockSpec((blk, IN_DIM), lambda i: (i, 0)),
                  pl.BlockSpec((IN_DIM, OUT), lambda i: (0, 0))],
        out_specs=pl.BlockSpec((blk, OUT), lambda i: (i, 0)),
        out_shape=jax.ShapeDtypeStruct((NPAD, OUT), jnp.float32),
    )(xp, W1)


def _fin_body(cat_ref, w_ref, b_ref, o_ref):
    j = pl.program_id(1)

    @pl.when(j == 0)
    def _():
        o_ref[...] = jnp.broadcast_to(b_ref[...], o_ref.shape)

    o_ref[...] += jnp.dot(cat_ref[0, 0], w_ref[0, 0],
                          preferred_element_type=jnp.float32)


def _final(cat6, WcT6, bc2):
    blk = 2048
    return pl.pallas_call(
        _fin_body,
        grid=(NPAD // blk, 6),
        in_specs=[pl.BlockSpec((1, 1, blk, H), lambda i, j: (j // 2, j % 2, i, 0)),
                  pl.BlockSpec((1, 1, H, OUT), lambda i, j: (j // 2, j % 2, 0, 0)),
                  pl.BlockSpec((1, OUT), lambda i, j: (0, 0))],
        out_specs=pl.BlockSpec((blk, OUT), lambda i, j: (i, 0)),
        out_shape=jax.ShapeDtypeStruct((NPAD, OUT), jnp.float32),
    )(cat6, WcT6, bc2)


def kernel(x, edge_index, edge_in, in_w, edge_out, out_w, W1, bias1, Wc, bc):
    xp = jnp.pad(x, ((0, NPAD - N), (0, 0)))
    h = _mm1(xp, W1)
    h2 = jnp.stack([h[:, :H], h[:, H:]])

    ez = jnp.zeros((EPAD - E,), jnp.int32)
    wz = jnp.zeros((EPAD - E,), jnp.float32)
    rall = jnp.concatenate([edge_index[0], ez, edge_in[0], ez, edge_out[0], ez])
    call = jnp.concatenate([edge_index[1], ez, edge_in[1], ez, edge_out[1], ez])
    wall = jnp.concatenate([jnp.ones((E,), jnp.float32), wz, in_w, wz, out_w, wz])

    cat6, _ = _sc_call(h2, rall, call, wall)

    WcT6 = Wc.T.reshape(3, 2, H, OUT)
    bc2 = (bc + jnp.tile(bias1[0], 3) @ Wc.T)[None, :]
    out = _final(cat6, WcT6, bc2)
    return out[:N]


# speedup vs baseline: 33.6200x; 1.1267x over previous
import functools

import jax
import jax.numpy as jnp
from jax import lax
from jax.experimental import pallas as pl
from jax.experimental.pallas import tpu as pltpu
from jax.experimental.pallas import tpu_sc as plsc

N = 100000
E = 1600000
IN_DIM = 128
OUT = 32
H = 16
NS = 16
NPAD = 100352
RPT = NPAD // NS
RCH = 128
ECH = 256
NCH = 392
EPAD = NS * ECH * NCH
SUB = 128


def _rsqrt16(d):
    s = jnp.float32(0.25) * d + jnp.float32(1.0)
    for _ in range(8):
        s = jnp.float32(0.5) * (s + d / s)
    return jnp.float32(1.0) / s


def _sc_body(h2, rall, call, wall,
             cat6, g,
             acc, deg,
             ebuf_r, ebuf_c, ebuf_w,
             gidx2, sidx2, wsub2,
             grows2,
             dense, dslice,
             semstage, semg, sems, semd):
    c = lax.axis_index("c")
    s = lax.axis_index("s")
    row0 = s * RPT
    goff = c * NPAD
    NSUB = ECH // SUB

    def _issue_stage(st, b):
        pltpu.async_copy(rall.at[pl.ds(st, ECH)], ebuf_r.at[b], semstage)
        pltpu.async_copy(call.at[pl.ds(st, ECH)], ebuf_c.at[b], semstage)
        pltpu.async_copy(wall.at[pl.ds(st, ECH)], ebuf_w.at[b], semstage)

    def _drain_stage():
        pltpu.make_async_copy(rall.at[pl.ds(0, ECH)], ebuf_r.at[0], semstage).wait()
        pltpu.make_async_copy(call.at[pl.ds(0, ECH)], ebuf_c.at[0], semstage).wait()
        pltpu.make_async_copy(wall.at[pl.ds(0, ECH)], ebuf_w.at[0], semstage).wait()

    def _compute_idx(b, sub, need_gidx):
        for gq in range(SUB // 16):
            o = sub * SUB + gq * 16
            rv = ebuf_r[b, pl.ds(o, 16)]
            cv = ebuf_c[b, pl.ds(o, 16)]
            wv = ebuf_w[b, pl.ds(o, 16)]
            w1 = jnp.where(rv == cv, jnp.float32(0.0), wv)
            wsub2[b, sub, pl.ds(gq * 16, 16)] = w1
            sidx2[b, sub, pl.ds(gq * 16, 16)] = cv
            if need_gidx:
                gidx2[b, sub, pl.ds(gq * 16, 16)] = rv + goff

    def _drain_deg_scatters(p):
        for _ in range(NSUB):
            pltpu.make_async_copy(wsub2.at[0, 0], deg.at[sidx2.at[0, 0]],
                                  semd.at[p]).wait()

    def _drain_gathers(p):
        for _ in range(NSUB):
            pltpu.make_async_copy(g.at[gidx2.at[0, 0]], grows2.at[0, 0],
                                  semg.at[p]).wait()

    def _drain_acc_scatters(p):
        for _ in range(NSUB):
            pltpu.make_async_copy(grows2.at[0, 0], acc.at[sidx2.at[0, 0]],
                                  sems.at[p]).wait()

    def _set(k, _):
        ebase = k * EPAD + s * (ECH * NCH)

        for j in range(RCH // 16):
            dslice[pl.ds(j * 16, 16)] = jnp.full((16,), 1.0, jnp.float32)

        def _p0(q, _):
            pltpu.sync_copy(dslice, deg.at[pl.ds(row0 + q * RCH, RCH)])
            return 0
        lax.fori_loop(0, RPT // RCH, _p0, 0)
        plsc.subcore_barrier()

        _issue_stage(ebase, 0)

        def _p1(i, _):
            b = i % 2
            _drain_stage()

            @pl.when(i + 1 < NCH)
            def _():
                _issue_stage(ebase + (i + 1) * ECH, 1 - b)

            @pl.when(i >= 2)
            def _():
                _drain_deg_scatters(b)

            def _p1s(sub, _):
                _compute_idx(b, sub, False)
                pltpu.async_copy(wsub2.at[b, sub], deg.at[sidx2.at[b, sub]],
                                 semd.at[b], add=True)
                return 0
            lax.fori_loop(0, NSUB, _p1s, 0)
            return 0
        lax.fori_loop(0, NCH, _p1, 0)
        _drain_deg_scatters(0)
        _drain_deg_scatters(1)
        plsc.subcore_barrier()

        def _p2(q, _):
            r0 = row0 + q * RCH
            pltpu.sync_copy(deg.at[pl.ds(r0, RCH)], dslice)
            for j in range(RCH // 16):
                d = dslice[pl.ds(j * 16, 16)]
                dslice[pl.ds(j * 16, 16)] = _rsqrt16(d)
            pltpu.sync_copy(dslice, deg.at[pl.ds(r0, RCH)])
            pltpu.sync_copy(h2.at[c, pl.ds(r0, RCH), :], dense)

            def _rs(m, _):
                dv = dslice[pl.ds(m * 16, 16)]
                for j in range(16):
                    n = m * 16 + j
                    dense[n, :] = dense[n, :] * dv[j]
                return 0
            lax.fori_loop(0, RCH // 16, _rs, 0)
            pltpu.sync_copy(dense, g.at[pl.ds(goff + r0, RCH), :])
            pltpu.sync_copy(dense, acc.at[pl.ds(r0, RCH), :])
            return 0
        lax.fori_loop(0, RPT // RCH, _p2, 0)
        plsc.subcore_barrier()

        _issue_stage(ebase, 0)

        def _p3(i, _):
            b = i % 2

            @pl.when(i < NCH)
            def _():
                _drain_stage()

                @pl.when(i + 1 < NCH)
                def _():
                    _issue_stage(ebase + (i + 1) * ECH, 1 - b)

                @pl.when(i >= 2)
                def _():
                    _drain_acc_scatters(b)

                def _p3a(sub, _):
                    _compute_idx(b, sub, True)
                    pltpu.async_copy(g.at[gidx2.at[b, sub]],
                                     grows2.at[b, sub], semg.at[b])
                    return 0
                lax.fori_loop(0, NSUB, _p3a, 0)

            @pl.when(i > 0)
            def _():
                pb = 1 - b
                _drain_gathers(pb)

                def _p3b(sub, _):
                    def _sc(q, _):
                        wv = wsub2[pb, sub, pl.ds(q * 16, 16)]
                        for u in range(16):
                            brow = q * 16 + u
                            grows2[pb, sub, brow, :] = (
                                grows2[pb, sub, brow, :] * wv[u])
                        return 0
                    lax.fori_loop(0, SUB // 16, _sc, 0)
                    pltpu.async_copy(grows2.at[pb, sub],
                                     acc.at[sidx2.at[pb, sub]],
                                     sems.at[pb], add=True)
                    return 0
                lax.fori_loop(0, NSUB, _p3b, 0)
            return 0
        lax.fori_loop(0, NCH + 1, _p3, 0)
        _drain_acc_scatters(0)
        _drain_acc_scatters(1)
        plsc.subcore_barrier()

        def _p4(q, _):
            r0 = row0 + q * RCH
            pltpu.sync_copy(acc.at[pl.ds(r0, RCH), :], dense)
            pltpu.sync_copy(deg.at[pl.ds(r0, RCH)], dslice)

            def _rs(m, _):
                dv = dslice[pl.ds(m * 16, 16)]
                for j in range(16):
                    n = m * 16 + j
                    dense[n, :] = dense[n, :] * dv[j]
                return 0
            lax.fori_loop(0, RCH // 16, _rs, 0)
            pltpu.sync_copy(dense,
                            cat6.at[pl.ds(r0, RCH),
                                    pl.ds(k * 32 + c * H, H)])
            return 0
        lax.fori_loop(0, RPT // RCH, _p4, 0)
        plsc.subcore_barrier()
        return 0

    lax.fori_loop(0, 3, _set, 0)


def _sc_call(h2, rall, call, wall):
    mesh = plsc.VectorSubcoreMesh(core_axis_name="c", subcore_axis_name="s")
    f = pl.kernel(
        _sc_body,
        out_type=(jax.ShapeDtypeStruct((NPAD, 3 * OUT), jnp.float32),
                  jax.ShapeDtypeStruct((2 * NPAD, H), jnp.float32)),
        mesh=mesh,
        compiler_params=pltpu.CompilerParams(use_tc_tiling_on_sc=False),
        scratch_types=[
            pltpu.VMEM_SHARED((NPAD, H), jnp.float32),
            pltpu.VMEM_SHARED((NPAD,), jnp.float32),
            pltpu.VMEM((2, ECH), jnp.int32),
            pltpu.VMEM((2, ECH), jnp.int32),
            pltpu.VMEM((2, ECH), jnp.float32),
            pltpu.VMEM((2, ECH // SUB, SUB), jnp.int32),
            pltpu.VMEM((2, ECH // SUB, SUB), jnp.int32),
            pltpu.VMEM((2, ECH // SUB, SUB), jnp.float32),
            pltpu.VMEM((2, ECH // SUB, SUB, H), jnp.float32),
            pltpu.VMEM((RCH, H), jnp.float32),
            pltpu.VMEM((RCH,), jnp.float32),
            pltpu.SemaphoreType.DMA,
            pltpu.SemaphoreType.DMA((2,)),
            pltpu.SemaphoreType.DMA((2,)),
            pltpu.SemaphoreType.DMA((2,)),
        ],
    )
    return f(h2, rall, call, wall)


def _mm1_body(x_ref, w_ref, o_ref):
    o_ref[0] = jnp.dot(x_ref[...], w_ref[0],
                       preferred_element_type=jnp.float32)


def _mm1(xp, W1h):
    blk = 2048
    return pl.pallas_call(
        _mm1_body,
        grid=(2, NPAD // blk),
        in_specs=[pl.BlockSpec((blk, IN_DIM), lambda k, i: (i, 0)),
                  pl.BlockSpec((1, IN_DIM, H), lambda k, i: (k, 0, 0))],
        out_specs=pl.BlockSpec((1, blk, H), lambda k, i: (k, i, 0)),
        out_shape=jax.ShapeDtypeStruct((2, NPAD, H), jnp.float32),
    )(xp, W1h)


def _fin_body(cat_ref, w_ref, b_ref, o_ref):
    o_ref[...] = jnp.dot(cat_ref[...], w_ref[...],
                         preferred_element_type=jnp.float32) + b_ref[...]


def _final(cat, WcT, bc2):
    blk = 2048
    return pl.pallas_call(
        _fin_body,
        grid=(NPAD // blk,),
        in_specs=[pl.BlockSpec((blk, 3 * OUT), lambda i: (i, 0)),
                  pl.BlockSpec((3 * OUT, OUT), lambda i: (0, 0)),
                  pl.BlockSpec((1, OUT), lambda i: (0, 0))],
        out_specs=pl.BlockSpec((blk, OUT), lambda i: (i, 0)),
        out_shape=jax.ShapeDtypeStruct((NPAD, OUT), jnp.float32),
    )(cat, WcT, bc2)


def kernel(x, edge_index, edge_in, in_w, edge_out, out_w, W1, bias1, Wc, bc):
    xp = jnp.pad(x, ((0, NPAD - N), (0, 0)))
    W1h = W1.T.reshape(2, H, IN_DIM).transpose(0, 2, 1)
    h2 = _mm1(xp, W1h)

    ez = jnp.zeros((EPAD - E,), jnp.int32)
    wz = jnp.zeros((EPAD - E,), jnp.float32)
    rall = jnp.concatenate([edge_index[0], ez, edge_in[0], ez, edge_out[0], ez])
    call = jnp.concatenate([edge_index[1], ez, edge_in[1], ez, edge_out[1], ez])
    wall = jnp.concatenate([jnp.ones((E,), jnp.float32), wz, in_w, wz, out_w, wz])

    cat, _ = _sc_call(h2, rall, call, wall)

    bc2 = (bc + jnp.tile(bias1[0], 3) @ Wc.T)[None, :]
    out = _final(cat, Wc.T, bc2)
    return out[:N]


# speedup vs baseline: 35.4857x; 1.0555x over previous
import functools

import jax
import jax.numpy as jnp
from jax import lax
from jax.experimental import pallas as pl
from jax.experimental.pallas import tpu as pltpu
from jax.experimental.pallas import tpu_sc as plsc

N = 100000
E = 1600000
IN_DIM = 128
OUT = 32
H = 16
NS = 16
NPAD = 100352
RPT = NPAD // NS
RCH = 128
ECH = 256
NCH = 392
EPAD = NS * ECH * NCH
SUB = 128


def _rsqrt16(d):
    s = jnp.float32(0.25) * d + jnp.float32(1.0)
    for _ in range(8):
        s = jnp.float32(0.5) * (s + d / s)
    return jnp.float32(1.0) / s


def _sc_body(h2, rall, call, wall,
             cat6, g,
             acc, deg,
             ebuf_r, ebuf_c, ebuf_w,
             gidx2, sidx2, wsub2,
             grows2,
             dense2, dslice2, vones,
             semstage, semg, sems, semd, semdr, semdw):
    c = lax.axis_index("c")
    s = lax.axis_index("s")
    row0 = s * RPT
    goff = c * NPAD
    NSUB = ECH // SUB
    NQ = RPT // RCH

    for j in range(RCH // 16):
        vones[pl.ds(j * 16, 16)] = jnp.full((16,), 1.0, jnp.float32)

    def _issue_stage(st, b):
        pltpu.async_copy(rall.at[pl.ds(st, ECH)], ebuf_r.at[b], semstage)
        pltpu.async_copy(call.at[pl.ds(st, ECH)], ebuf_c.at[b], semstage)
        pltpu.async_copy(wall.at[pl.ds(st, ECH)], ebuf_w.at[b], semstage)

    def _drain_stage():
        pltpu.make_async_copy(rall.at[pl.ds(0, ECH)], ebuf_r.at[0], semstage).wait()
        pltpu.make_async_copy(call.at[pl.ds(0, ECH)], ebuf_c.at[0], semstage).wait()
        pltpu.make_async_copy(wall.at[pl.ds(0, ECH)], ebuf_w.at[0], semstage).wait()

    def _compute_idx(b, sub, need_gidx):
        for gq in range(SUB // 16):
            o = sub * SUB + gq * 16
            rv = ebuf_r[b, pl.ds(o, 16)]
            cv = ebuf_c[b, pl.ds(o, 16)]
            wv = ebuf_w[b, pl.ds(o, 16)]
            w1 = jnp.where(rv == cv, jnp.float32(0.0), wv)
            wsub2[b, sub, pl.ds(gq * 16, 16)] = w1
            sidx2[b, sub, pl.ds(gq * 16, 16)] = cv
            if need_gidx:
                gidx2[b, sub, pl.ds(gq * 16, 16)] = rv + goff

    def _drain_deg_scatters(p):
        for _ in range(NSUB):
            pltpu.make_async_copy(wsub2.at[0, 0], deg.at[sidx2.at[0, 0]],
                                  semd.at[p]).wait()

    def _drain_gathers(p):
        for _ in range(NSUB):
            pltpu.make_async_copy(g.at[gidx2.at[0, 0]], grows2.at[0, 0],
                                  semg.at[p]).wait()

    def _drain_acc_scatters(p):
        for _ in range(NSUB):
            pltpu.make_async_copy(grows2.at[0, 0], acc.at[sidx2.at[0, 0]],
                                  sems.at[p]).wait()

    def _set(k, _):
        ebase = k * EPAD + s * (ECH * NCH)

        def _p0(q, _):
            pltpu.sync_copy(vones, deg.at[pl.ds(row0 + q * RCH, RCH)])
            return 0
        lax.fori_loop(0, NQ, _p0, 0)
        plsc.subcore_barrier()

        _issue_stage(ebase, 0)

        def _p1(i, _):
            b = i % 2
            _drain_stage()

            @pl.when(i + 1 < NCH)
            def _():
                _issue_stage(ebase + (i + 1) * ECH, 1 - b)

            @pl.when(i >= 2)
            def _():
                _drain_deg_scatters(b)

            def _p1s(sub, _):
                _compute_idx(b, sub, False)
                pltpu.async_copy(wsub2.at[b, sub], deg.at[sidx2.at[b, sub]],
                                 semd.at[b], add=True)
                return 0
            lax.fori_loop(0, NSUB, _p1s, 0)
            return 0
        lax.fori_loop(0, NCH, _p1, 0)
        _drain_deg_scatters(0)
        _drain_deg_scatters(1)
        plsc.subcore_barrier()

        pltpu.async_copy(h2.at[c, pl.ds(row0, RCH), :], dense2.at[0], semdr)

        def _p2(q, _):
            b = q % 2
            pltpu.make_async_copy(h2.at[0, pl.ds(0, RCH), :], dense2.at[0],
                                  semdr).wait()

            @pl.when(q >= 1)
            def _():
                pltpu.make_async_copy(dense2.at[0], g.at[pl.ds(0, RCH), :],
                                      semdw).wait()

            @pl.when(q + 1 < NQ)
            def _():
                pltpu.async_copy(h2.at[c, pl.ds(row0 + (q + 1) * RCH, RCH), :],
                                 dense2.at[1 - b], semdr)

            r0 = row0 + q * RCH
            pltpu.sync_copy(deg.at[pl.ds(r0, RCH)], dslice2.at[b])
            for j in range(RCH // 16):
                d = dslice2[b, pl.ds(j * 16, 16)]
                dslice2[b, pl.ds(j * 16, 16)] = _rsqrt16(d)
            pltpu.sync_copy(dslice2.at[b], deg.at[pl.ds(r0, RCH)])

            def _rs(m, _):
                dv = dslice2[b, pl.ds(m * 16, 16)]
                for j in range(16):
                    n = m * 16 + j
                    dense2[b, n, :] = dense2[b, n, :] * dv[j]
                return 0
            lax.fori_loop(0, RCH // 16, _rs, 0)

            pltpu.async_copy(dense2.at[b], g.at[pl.ds(goff + r0, RCH), :],
                             semdw)
            pltpu.sync_copy(dense2.at[b], acc.at[pl.ds(r0, RCH), :])
            return 0
        lax.fori_loop(0, NQ, _p2, 0)
        pltpu.make_async_copy(dense2.at[0], g.at[pl.ds(0, RCH), :],
                              semdw).wait()
        plsc.subcore_barrier()

        _issue_stage(ebase, 0)

        def _p3(i, _):
            b = i % 2

            @pl.when(i < NCH)
            def _():
                _drain_stage()

                @pl.when(i + 1 < NCH)
                def _():
                    _issue_stage(ebase + (i + 1) * ECH, 1 - b)

                @pl.when(i >= 2)
                def _():
                    _drain_acc_scatters(b)

                def _p3a(sub, _):
                    _compute_idx(b, sub, True)
                    pltpu.async_copy(g.at[gidx2.at[b, sub]],
                                     grows2.at[b, sub], semg.at[b])
                    return 0
                lax.fori_loop(0, NSUB, _p3a, 0)

            @pl.when(i > 0)
            def _():
                pb = 1 - b
                _drain_gathers(pb)

                def _p3b(sub, _):
                    def _sc(q, _):
                        wv = wsub2[pb, sub, pl.ds(q * 16, 16)]
                        for u in range(16):
                            brow = q * 16 + u
                            grows2[pb, sub, brow, :] = (
                                grows2[pb, sub, brow, :] * wv[u])
                        return 0
                    lax.fori_loop(0, SUB // 16, _sc, 0)
                    pltpu.async_copy(grows2.at[pb, sub],
                                     acc.at[sidx2.at[pb, sub]],
                                     sems.at[pb], add=True)
                    return 0
                lax.fori_loop(0, NSUB, _p3b, 0)
            return 0
        lax.fori_loop(0, NCH + 1, _p3, 0)
        _drain_acc_scatters(0)
        _drain_acc_scatters(1)
        plsc.subcore_barrier()

        def _p4(q, _):
            r0 = row0 + q * RCH
            pltpu.sync_copy(acc.at[pl.ds(r0, RCH), :], dense2.at[0])
            pltpu.sync_copy(deg.at[pl.ds(r0, RCH)], dslice2.at[0])

            def _rs(m, _):
                dv = dslice2[0, pl.ds(m * 16, 16)]
                for j in range(16):
                    n = m * 16 + j
                    dense2[0, n, :] = dense2[0, n, :] * dv[j]
                return 0
            lax.fori_loop(0, RCH // 16, _rs, 0)
            pltpu.sync_copy(dense2.at[0],
                            cat6.at[pl.ds(r0, RCH),
                                    pl.ds(k * 32 + c * H, H)])
            return 0
        lax.fori_loop(0, NQ, _p4, 0)
        plsc.subcore_barrier()
        return 0

    lax.fori_loop(0, 3, _set, 0)


def _sc_call(h2, rall, call, wall):
    mesh = plsc.VectorSubcoreMesh(core_axis_name="c", subcore_axis_name="s")
    f = pl.kernel(
        _sc_body,
        out_type=(jax.ShapeDtypeStruct((NPAD, 3 * OUT), jnp.float32),
                  jax.ShapeDtypeStruct((2 * NPAD, H), jnp.float32)),
        mesh=mesh,
        compiler_params=pltpu.CompilerParams(use_tc_tiling_on_sc=False),
        scratch_types=[
            pltpu.VMEM_SHARED((NPAD, H), jnp.float32),
            pltpu.VMEM_SHARED((NPAD,), jnp.float32),
            pltpu.VMEM((2, ECH), jnp.int32),
            pltpu.VMEM((2, ECH), jnp.int32),
            pltpu.VMEM((2, ECH), jnp.float32),
            pltpu.VMEM((2, ECH // SUB, SUB), jnp.int32),
            pltpu.VMEM((2, ECH // SUB, SUB), jnp.int32),
            pltpu.VMEM((2, ECH // SUB, SUB), jnp.float32),
            pltpu.VMEM((2, ECH // SUB, SUB, H), jnp.float32),
            pltpu.VMEM((2, RCH, H), jnp.float32),
            pltpu.VMEM((2, RCH), jnp.float32),
            pltpu.VMEM((RCH,), jnp.float32),
            pltpu.SemaphoreType.DMA,
            pltpu.SemaphoreType.DMA((2,)),
            pltpu.SemaphoreType.DMA((2,)),
            pltpu.SemaphoreType.DMA((2,)),
            pltpu.SemaphoreType.DMA,
            pltpu.SemaphoreType.DMA,
        ],
    )
    return f(h2, rall, call, wall)


def _mm1_body(x_ref, w_ref, o_ref):
    x = x_ref[...]
    o_ref[0] = jnp.dot(x, w_ref[0], preferred_element_type=jnp.float32)
    o_ref[1] = jnp.dot(x, w_ref[1], preferred_element_type=jnp.float32)


def _mm1(xp, W1h):
    blk = 2048
    return pl.pallas_call(
        _mm1_body,
        grid=(NPAD // blk,),
        in_specs=[pl.BlockSpec((blk, IN_DIM), lambda i: (i, 0)),
                  pl.BlockSpec((2, IN_DIM, H), lambda i: (0, 0, 0))],
        out_specs=pl.BlockSpec((2, blk, H), lambda i: (0, i, 0)),
        out_shape=jax.ShapeDtypeStruct((2, NPAD, H), jnp.float32),
    )(xp, W1h)


def _fin_body(cat_ref, w_ref, b_ref, o_ref):
    o_ref[...] = jnp.dot(cat_ref[...], w_ref[...],
                         preferred_element_type=jnp.float32) + b_ref[...]


def _final(cat, WcT, bc2):
    blk = 2048
    return pl.pallas_call(
        _fin_body,
        grid=(NPAD // blk,),
        in_specs=[pl.BlockSpec((blk, 3 * OUT), lambda i: (i, 0)),
                  pl.BlockSpec((3 * OUT, OUT), lambda i: (0, 0)),
                  pl.BlockSpec((1, OUT), lambda i: (0, 0))],
        out_specs=pl.BlockSpec((blk, OUT), lambda i: (i, 0)),
        out_shape=jax.ShapeDtypeStruct((NPAD, OUT), jnp.float32),
    )(cat, WcT, bc2)


def kernel(x, edge_index, edge_in, in_w, edge_out, out_w, W1, bias1, Wc, bc):
    xp = jnp.pad(x, ((0, NPAD - N), (0, 0)))
    W1h = W1.T.reshape(2, H, IN_DIM).transpose(0, 2, 1)
    h2 = _mm1(xp, W1h)

    ez = jnp.zeros((EPAD - E,), jnp.int32)
    wz = jnp.zeros((EPAD - E,), jnp.float32)
    rall = jnp.concatenate([edge_index[0], ez, edge_in[0], ez, edge_out[0], ez])
    call = jnp.concatenate([edge_index[1], ez, edge_in[1], ez, edge_out[1], ez])
    wall = jnp.concatenate([jnp.ones((E,), jnp.float32), wz, in_w, wz, out_w, wz])

    cat, _ = _sc_call(h2, rall, call, wall)

    bc2 = (bc + jnp.tile(bias1[0], 3) @ Wc.T)[None, :]
    out = _final(cat, Wc.T, bc2)
    return out[:N]


# speedup vs baseline: 35.8477x; 1.0102x over previous
import functools

import jax
import jax.numpy as jnp
from jax import lax
from jax.experimental import pallas as pl
from jax.experimental.pallas import tpu as pltpu
from jax.experimental.pallas import tpu_sc as plsc

N = 100000
E = 1600000
IN_DIM = 128
OUT = 32
H = 16
NS = 16
NPAD = 100352
RPT = NPAD // NS
RCH = 128
ECH = 256
NCH = 392
EPAD = NS * ECH * NCH
SUB = 128


def _rsqrt16(d):
    s = jnp.float32(0.25) * d + jnp.float32(1.0)
    for _ in range(8):
        s = jnp.float32(0.5) * (s + d / s)
    return jnp.float32(1.0) / s


def _sc_body(h2, rall, call, wall,
             cat6, g,
             acc, deg,
             ebuf_r, ebuf_c, ebuf_w,
             gidx2, sidx2, wsub2,
             grows2,
             dense2, dslice2, vones,
             semstage, semg, sems, semd, semdr, semdw):
    c = lax.axis_index("c")
    s = lax.axis_index("s")
    row0 = s * RPT
    goff = c * NPAD
    NSUB = ECH // SUB
    NQ = RPT // RCH

    for j in range(RCH // 16):
        vones[pl.ds(j * 16, 16)] = jnp.full((16,), 1.0, jnp.float32)

    def _issue_stage(st, b):
        pltpu.async_copy(rall.at[pl.ds(st, ECH)], ebuf_r.at[b], semstage)
        pltpu.async_copy(call.at[pl.ds(st, ECH)], ebuf_c.at[b], semstage)
        pltpu.async_copy(wall.at[pl.ds(st, ECH)], ebuf_w.at[b], semstage)

    def _drain_stage():
        pltpu.make_async_copy(rall.at[pl.ds(0, ECH)], ebuf_r.at[0], semstage).wait()
        pltpu.make_async_copy(call.at[pl.ds(0, ECH)], ebuf_c.at[0], semstage).wait()
        pltpu.make_async_copy(wall.at[pl.ds(0, ECH)], ebuf_w.at[0], semstage).wait()

    def _compute_idx(b, sub, need_gidx):
        for gq in range(SUB // 16):
            o = sub * SUB + gq * 16
            rv = ebuf_r[b, pl.ds(o, 16)]
            cv = ebuf_c[b, pl.ds(o, 16)]
            wv = ebuf_w[b, pl.ds(o, 16)]
            w1 = jnp.where(rv == cv, jnp.float32(0.0), wv)
            wsub2[b, sub, pl.ds(gq * 16, 16)] = w1
            sidx2[b, sub, pl.ds(gq * 16, 16)] = cv
            if need_gidx:
                gidx2[b, sub, pl.ds(gq * 16, 16)] = rv + goff

    def _drain_deg_scatters(p):
        for _ in range(NSUB):
            pltpu.make_async_copy(wsub2.at[0, 0], deg.at[sidx2.at[0, 0]],
                                  semd.at[p]).wait()

    def _drain_gathers(p):
        for _ in range(NSUB):
            pltpu.make_async_copy(g.at[gidx2.at[0, 0]], grows2.at[0, 0],
                                  semg.at[p]).wait()

    def _drain_acc_scatters(p):
        for _ in range(NSUB):
            pltpu.make_async_copy(grows2.at[0, 0], acc.at[sidx2.at[0, 0]],
                                  sems.at[p]).wait()

    def _set(k, _):
        ebase = k * EPAD + s * (ECH * NCH)

        def _p0(q, _):
            pltpu.sync_copy(vones, deg.at[pl.ds(row0 + q * RCH, RCH)])
            return 0
        lax.fori_loop(0, NQ, _p0, 0)
        plsc.subcore_barrier()

        _issue_stage(ebase, 0)

        def _p1(i, _):
            b = i % 2
            _drain_stage()

            @pl.when(i + 1 < NCH)
            def _():
                _issue_stage(ebase + (i + 1) * ECH, 1 - b)

            @pl.when(i >= 2)
            def _():
                _drain_deg_scatters(b)

            def _p1s(sub, _):
                _compute_idx(b, sub, False)
                pltpu.async_copy(wsub2.at[b, sub], deg.at[sidx2.at[b, sub]],
                                 semd.at[b], add=True)
                return 0
            lax.fori_loop(0, NSUB, _p1s, 0)
            return 0
        lax.fori_loop(0, NCH, _p1, 0)
        _drain_deg_scatters(0)
        _drain_deg_scatters(1)
        plsc.subcore_barrier()

        pltpu.async_copy(h2.at[c, pl.ds(row0, RCH), :], dense2.at[0], semdr)

        def _p2(q, _):
            b = q % 2
            pltpu.make_async_copy(h2.at[0, pl.ds(0, RCH), :], dense2.at[0],
                                  semdr).wait()

            @pl.when(q >= 1)
            def _():
                pltpu.make_async_copy(dense2.at[0], g.at[pl.ds(0, RCH), :],
                                      semdw).wait()

            @pl.when(q + 1 < NQ)
            def _():
                pltpu.async_copy(h2.at[c, pl.ds(row0 + (q + 1) * RCH, RCH), :],
                                 dense2.at[1 - b], semdr)

            r0 = row0 + q * RCH
            pltpu.sync_copy(deg.at[pl.ds(r0, RCH)], dslice2.at[b])
            for j in range(RCH // 16):
                d = dslice2[b, pl.ds(j * 16, 16)]
                dslice2[b, pl.ds(j * 16, 16)] = _rsqrt16(d)
            pltpu.sync_copy(dslice2.at[b], deg.at[pl.ds(r0, RCH)])

            def _rs(m, _):
                dv = dslice2[b, pl.ds(m * 16, 16)]
                for j in range(16):
                    n = m * 16 + j
                    dense2[b, n, :] = dense2[b, n, :] * dv[j]
                return 0
            lax.fori_loop(0, RCH // 16, _rs, 0)

            pltpu.async_copy(dense2.at[b], g.at[pl.ds(goff + r0, RCH), :],
                             semdw)
            pltpu.sync_copy(dense2.at[b], acc.at[pl.ds(r0, RCH), :])
            return 0
        lax.fori_loop(0, NQ, _p2, 0)
        pltpu.make_async_copy(dense2.at[0], g.at[pl.ds(0, RCH), :],
                              semdw).wait()
        plsc.subcore_barrier()

        _issue_stage(ebase, 0)

        def _p3(i, _):
            b = i % 2

            @pl.when(i < NCH)
            def _():
                _drain_stage()

                @pl.when(i + 1 < NCH)
                def _():
                    _issue_stage(ebase + (i + 1) * ECH, 1 - b)

                @pl.when(i >= 2)
                def _():
                    _drain_acc_scatters(b)

                def _p3a(sub, _):
                    _compute_idx(b, sub, True)
                    pltpu.async_copy(g.at[gidx2.at[b, sub]],
                                     grows2.at[b, sub], semg.at[b])
                    return 0
                lax.fori_loop(0, NSUB, _p3a, 0)

            @pl.when(i > 0)
            def _():
                pb = 1 - b
                _drain_gathers(pb)

                def _p3b(sub, _):
                    def _sc(q, _):
                        wv = wsub2[pb, sub, pl.ds(q * 16, 16)]
                        for u in range(16):
                            brow = q * 16 + u
                            grows2[pb, sub, brow, :] = (
                                grows2[pb, sub, brow, :] * wv[u])
                        return 0
                    lax.fori_loop(0, SUB // 16, _sc, 0)
                    pltpu.async_copy(grows2.at[pb, sub],
                                     acc.at[sidx2.at[pb, sub]],
                                     sems.at[pb], add=True)
                    return 0
                lax.fori_loop(0, NSUB, _p3b, 0)
            return 0
        lax.fori_loop(0, NCH + 1, _p3, 0)
        _drain_acc_scatters(0)
        _drain_acc_scatters(1)
        plsc.subcore_barrier()

        def _p4(q, _):
            b = q % 2
            r0 = row0 + q * RCH

            @pl.when(q >= 2)
            def _():
                pltpu.make_async_copy(dense2.at[0],
                                      cat6.at[pl.ds(0, RCH), pl.ds(0, H)],
                                      semdw).wait()

            pltpu.sync_copy(acc.at[pl.ds(r0, RCH), :], dense2.at[b])
            pltpu.sync_copy(deg.at[pl.ds(r0, RCH)], dslice2.at[b])

            def _rs(m, _):
                dv = dslice2[b, pl.ds(m * 16, 16)]
                for j in range(16):
                    n = m * 16 + j
                    dense2[b, n, :] = dense2[b, n, :] * dv[j]
                return 0
            lax.fori_loop(0, RCH // 16, _rs, 0)
            pltpu.async_copy(dense2.at[b],
                             cat6.at[pl.ds(r0, RCH),
                                     pl.ds(k * 32 + c * H, H)], semdw)
            return 0
        lax.fori_loop(0, NQ, _p4, 0)
        pltpu.make_async_copy(dense2.at[0],
                              cat6.at[pl.ds(0, RCH), pl.ds(0, H)],
                              semdw).wait()
        pltpu.make_async_copy(dense2.at[0],
                              cat6.at[pl.ds(0, RCH), pl.ds(0, H)],
                              semdw).wait()
        plsc.subcore_barrier()
        return 0

    lax.fori_loop(0, 3, _set, 0)


def _sc_call(h2, rall, call, wall):
    mesh = plsc.VectorSubcoreMesh(core_axis_name="c", subcore_axis_name="s")
    f = pl.kernel(
        _sc_body,
        out_type=(jax.ShapeDtypeStruct((NPAD, 3 * OUT), jnp.float32),
                  jax.ShapeDtypeStruct((2 * NPAD, H), jnp.float32)),
        mesh=mesh,
        compiler_params=pltpu.CompilerParams(use_tc_tiling_on_sc=False),
        scratch_types=[
            pltpu.VMEM_SHARED((NPAD, H), jnp.float32),
            pltpu.VMEM_SHARED((NPAD,), jnp.float32),
            pltpu.VMEM((2, ECH), jnp.int32),
            pltpu.VMEM((2, ECH), jnp.int32),
            pltpu.VMEM((2, ECH), jnp.float32),
            pltpu.VMEM((2, ECH // SUB, SUB), jnp.int32),
            pltpu.VMEM((2, ECH // SUB, SUB), jnp.int32),
            pltpu.VMEM((2, ECH // SUB, SUB), jnp.float32),
            pltpu.VMEM((2, ECH // SUB, SUB, H), jnp.float32),
            pltpu.VMEM((2, RCH, H), jnp.float32),
            pltpu.VMEM((2, RCH), jnp.float32),
            pltpu.VMEM((RCH,), jnp.float32),
            pltpu.SemaphoreType.DMA,
            pltpu.SemaphoreType.DMA((2,)),
            pltpu.SemaphoreType.DMA((2,)),
            pltpu.SemaphoreType.DMA((2,)),
            pltpu.SemaphoreType.DMA,
            pltpu.SemaphoreType.DMA,
        ],
    )
    return f(h2, rall, call, wall)


def _mm1_body(x_ref, w_ref, o_ref):
    x = x_ref[...]
    o_ref[0] = jnp.dot(x, w_ref[0], preferred_element_type=jnp.float32)
    o_ref[1] = jnp.dot(x, w_ref[1], preferred_element_type=jnp.float32)


def _mm1(xp, W1h):
    blk = 2048
    return pl.pallas_call(
        _mm1_body,
        grid=(NPAD // blk,),
        in_specs=[pl.BlockSpec((blk, IN_DIM), lambda i: (i, 0)),
                  pl.BlockSpec((2, IN_DIM, H), lambda i: (0, 0, 0))],
        out_specs=pl.BlockSpec((2, blk, H), lambda i: (0, i, 0)),
        out_shape=jax.ShapeDtypeStruct((2, NPAD, H), jnp.float32),
    )(xp, W1h)


def _fin_body(cat_ref, w_ref, b_ref, o_ref):
    o_ref[...] = jnp.dot(cat_ref[...], w_ref[...],
                         preferred_element_type=jnp.float32) + b_ref[...]


def _final(cat, WcT, bc2):
    blk = 2048
    return pl.pallas_call(
        _fin_body,
        grid=(NPAD // blk,),
        in_specs=[pl.BlockSpec((blk, 3 * OUT), lambda i: (i, 0)),
                  pl.BlockSpec((3 * OUT, OUT), lambda i: (0, 0)),
                  pl.BlockSpec((1, OUT), lambda i: (0, 0))],
        out_specs=pl.BlockSpec((blk, OUT), lambda i: (i, 0)),
        out_shape=jax.ShapeDtypeStruct((NPAD, OUT), jnp.float32),
    )(cat, WcT, bc2)


def kernel(x, edge_index, edge_in, in_w, edge_out, out_w, W1, bias1, Wc, bc):
    xp = jnp.pad(x, ((0, NPAD - N), (0, 0)))
    W1h = W1.T.reshape(2, H, IN_DIM).transpose(0, 2, 1)
    h2 = _mm1(xp, W1h)

    ez = jnp.zeros((EPAD - E,), jnp.int32)
    wz = jnp.zeros((EPAD - E,), jnp.float32)
    rall = jnp.concatenate([edge_index[0], ez, edge_in[0], ez, edge_out[0], ez])
    call = jnp.concatenate([edge_index[1], ez, edge_in[1], ez, edge_out[1], ez])
    wall = jnp.concatenate([jnp.ones((E,), jnp.float32), wz, in_w, wz, out_w, wz])

    cat, _ = _sc_call(h2, rall, call, wall)

    bc2 = (bc + jnp.tile(bias1[0], 3) @ Wc.T)[None, :]
    out = _final(cat, Wc.T, bc2)
    return out[:N]
